# baseline (speedup 1.0000x reference)
import functools

import jax
import jax.numpy as jnp
from jax import lax
from jax.experimental import pallas as pl
from jax.experimental.pallas import tpu as pltpu
from jax.experimental.pallas import tpu_sc as plsc

N = 10000
D = 256
E = 160000
ITERS = 3

NTILES = 32
BROWS = 320
TRASH = 320
AGG_ROWS = 321

CHUNK = 128
HCH = CHUNK // 2
PC = 1280
NPC = E // PC
CCAP = CHUNK + 16
DATA0 = 128
LIST_LEN = DATA0 + E + CHUNK


def _partition(src, dst):
    mesh = plsc.VectorSubcoreMesh(core_axis_name="c", subcore_axis_name="s")

    @functools.partial(
        pl.kernel,
        out_type=[
            jax.ShapeDtypeStruct((NTILES, 1, LIST_LEN), jnp.int32),
            jax.ShapeDtypeStruct((NTILES, 1, LIST_LEN), jnp.int32),
        ],
        mesh=mesh,
        compiler_params=pltpu.CompilerParams(needs_layout_passes=False),
        scratch_types=[
            pltpu.VMEM((PC,), jnp.int32),
            pltpu.VMEM((PC,), jnp.int32),
            pltpu.VMEM((PC,), jnp.int32),
            pltpu.VMEM((PC,), jnp.int32),
            pltpu.VMEM((CCAP,), jnp.int32),
            pltpu.VMEM((CCAP,), jnp.int32),
            pltpu.VMEM((16,), jnp.int32),
            pltpu.SMEM((4,), jnp.int32),
            pltpu.SemaphoreType.DMA,
            pltpu.SemaphoreType.DMA,
            pltpu.SemaphoreType.DMA,
            pltpu.SemaphoreType.DMA,
        ],
    )
    def part(src_hbm, dst_hbm, ls_hbm, ld_hbm,
             sv0, sv1, dv0, dv1, cs, cd, hv, st,
             ss0, ss1, ds0, ds1):
        c = lax.axis_index("c")
        s = lax.axis_index("s")
        w = c * 16 + s
        lo = w * BROWS
        svr = (sv0, sv1)
        dvr = (dv0, dv1)
        ssem = (ss0, ss1)
        dsem = (ds0, ds1)
        st[0] = 0
        st[1] = DATA0
        st[2] = 0

        def fire_scan(k, p):
            pltpu.async_copy(src_hbm.at[pl.ds(k * PC, PC)], svr[p], ssem[p])
            pltpu.async_copy(dst_hbm.at[pl.ds(k * PC, PC)], dvr[p], dsem[p])

        fire_scan(0, 0)

        def scan_chunk(k, p):
            sv = svr[p]
            dv = dvr[p]
            pltpu.make_async_copy(src_hbm.at[pl.ds(0, PC)], sv, ssem[p]).wait()
            pltpu.make_async_copy(dst_hbm.at[pl.ds(0, PC)], dv, dsem[p]).wait()

            @pl.when(k + 1 < NPC)
            def _():
                fire_scan(k + 1, 1 - p)

            for g in range(PC // 16):
                d16 = dv[pl.ds(g * 16, 16)]
                bkt = lax.shift_right_logical(d16 * 6554, 21)
                m = bkt == w
                npc = plsc.all_reduce_population_count(m)[0]

                @pl.when(npc > 0)
                def _():
                    off = st[0]
                    s16 = sv[pl.ds(g * 16, 16)]
                    plsc.store_compressed(cs.at[pl.ds(off, 16)], s16, mask=m)
                    plsc.store_compressed(cd.at[pl.ds(off, 16)], d16 - lo, mask=m)
                    st[0] = off + npc

                @pl.when(st[0] >= CHUNK)
                def _():
                    pos = pl.multiple_of(st[1], CHUNK)
                    pltpu.sync_copy(cs.at[pl.ds(0, CHUNK)],
                                    ls_hbm.at[w, 0, pl.ds(pos, CHUNK)])
                    pltpu.sync_copy(cd.at[pl.ds(0, CHUNK)],
                                    ld_hbm.at[w, 0, pl.ds(pos, CHUNK)])
                    ts = cs[pl.ds(CHUNK, 16)]
                    cs[pl.ds(0, 16)] = ts
                    td = cd[pl.ds(CHUNK, 16)]
                    cd[pl.ds(0, 16)] = td
                    st[0] = st[0] - CHUNK
                    st[1] = pos + CHUNK
                    st[2] = st[2] + 1

        def outer(kk, carry):
            scan_chunk(kk * 2, 0)

            @pl.when(kk * 2 + 1 < NPC)
            def _():
                scan_chunk(kk * 2 + 1, 1)

            return carry

        lax.fori_loop(0, (NPC + 1) // 2, outer, 0)

        off = st[0]
        iota = lax.iota(jnp.int32, 16)
        for g in range(CCAP // 16):
            keep = (iota + g * 16) < off
            vs = cs[pl.ds(g * 16, 16)]
            cs[pl.ds(g * 16, 16)] = jnp.where(keep, vs, 0)
            vd = cd[pl.ds(g * 16, 16)]
            cd[pl.ds(g * 16, 16)] = jnp.where(keep, vd, TRASH)
        pos = pl.multiple_of(st[1], CHUNK)
        pltpu.sync_copy(cs.at[pl.ds(0, CHUNK)],
                        ls_hbm.at[w, 0, pl.ds(pos, CHUNK)])
        pltpu.sync_copy(cd.at[pl.ds(0, CHUNK)],
                        ld_hbm.at[w, 0, pl.ds(pos, CHUNK)])
        nch = st[2] + 1
        hv[pl.ds(0, 16)] = jnp.full((16,), nch, jnp.int32)
        pltpu.sync_copy(hv.at[pl.ds(0, 8)], ls_hbm.at[w, 0, pl.ds(0, 8)])

    return part(src, dst)


def _seg_sum(h, ls, ld, zeros):
    mesh = plsc.VectorSubcoreMesh(core_axis_name="c", subcore_axis_name="s")

    @functools.partial(
        pl.kernel,
        out_type=jax.ShapeDtypeStruct((N, D), jnp.float32),
        mesh=mesh,
        compiler_params=pltpu.CompilerParams(needs_layout_passes=False),
        scratch_types=[
            pltpu.VMEM((16,), jnp.int32),
            pltpu.VMEM((1, CHUNK), jnp.int32),
            pltpu.VMEM((1, CHUNK), jnp.int32),
            pltpu.VMEM((CHUNK,), jnp.int32),
            pltpu.VMEM((CHUNK,), jnp.int32),
            pltpu.VMEM((CHUNK,), jnp.int32),
            pltpu.VMEM((HCH, D // 2), jnp.int32),
            pltpu.VMEM((HCH, D // 2), jnp.int32),
            pltpu.VMEM((AGG_ROWS, D), jnp.float32),
            pltpu.SemaphoreType.DMA,
            pltpu.SemaphoreType.DMA,
            pltpu.SemaphoreType.DMA,
            pltpu.SemaphoreType.DMA,
            pltpu.SemaphoreType.DMA,
            pltpu.SemaphoreType.DMA,
        ],
    )
    def acc(h_hbm, ls_hbm, ld_hbm, z_hbm, out_hbm,
            hv, sv0, sv1, dlv0, dlv1, dlw, ra, rb, agg,
            is0, is1, id0, id1, gsa, gsb):
        c = lax.axis_index("c")
        s = lax.axis_index("s")
        w = c * 16 + s
        sv = (sv0, sv1)
        dlv = (dlv0, dlv1)
        isem_s = (is0, is1)
        isem_d = (id0, id1)
        pltpu.sync_copy(z_hbm, agg)
        pltpu.sync_copy(ls_hbm.at[w, 0, pl.ds(0, 8)], hv.at[pl.ds(0, 8)])
        nch = hv[pl.ds(0, 16)][0]

        def fire_idx(k, p):
            base = pl.multiple_of(DATA0 + k * CHUNK, CHUNK)
            pltpu.async_copy(ls_hbm.at[w, 0, pl.ds(base, CHUNK)],
                             sv[p].at[0], isem_s[p])
            pltpu.async_copy(ld_hbm.at[w, 0, pl.ds(base, CHUNK)],
                             dlv[p], isem_d[p])

        def wait_idx_s(p):
            pltpu.make_async_copy(ls_hbm.at[w, 0, pl.ds(0, CHUNK)],
                                  sv[p].at[0], isem_s[p]).wait()

        def fire_gather_a(p):
            pltpu.async_copy(h_hbm.at[sv[p].at[0, pl.ds(0, HCH)]], ra, gsa)

        fire_idx(0, 0)

        @pl.when(nch > 1)
        def _():
            fire_idx(1, 1)

        wait_idx_s(0)
        fire_gather_a(0)

        def accum_half(rbuf, mbase):
            def edge_group(m, carry2):
                dl16 = dlw[pl.ds(mbase * 16 + m * 16, 16)]
                for i in range(0, 16, 2):
                    dla = dl16[i]
                    dlb = dl16[i + 1]
                    va = []
                    vb = []
                    for j in range(D // 32):
                        pa = plsc.bitcast(rbuf[m * 16 + i, pl.ds(j * 16, 16)],
                                          jnp.bfloat16)
                        va.append(plsc.unpack(pa, format=plsc.PackFormat.INTERLEAVED))
                    for j in range(D // 32):
                        pb = plsc.bitcast(rbuf[m * 16 + i + 1, pl.ds(j * 16, 16)],
                                          jnp.bfloat16)
                        vb.append(plsc.unpack(pb, format=plsc.PackFormat.INTERLEAVED))
                    for j in range(D // 32):
                        plsc.addupdate(agg.at[dla, pl.ds(j * 32, 16)], va[j][0])
                        plsc.addupdate(agg.at[dla, pl.ds(j * 32 + 16, 16)], va[j][1])
                    for j in range(D // 32):
                        plsc.addupdate(agg.at[dlb, pl.ds(j * 32, 16)], vb[j][0])
                        plsc.addupdate(agg.at[dlb, pl.ds(j * 32 + 16, 16)], vb[j][1])
                return carry2

            lax.fori_loop(0, HCH // 16, edge_group, 0)

        def body(k, p):
            pltpu.make_async_copy(h_hbm.at[sv[p].at[0, pl.ds(0, HCH)]],
                                  ra, gsa).wait()
            pltpu.make_async_copy(ld_hbm.at[w, 0, pl.ds(0, CHUNK)],
                                  dlv[p], isem_d[p]).wait()
            for m in range(CHUNK // 16):
                dlw[pl.ds(m * 16, 16)] = dlv[p][pl.ds(m * 16, 16)]
            pltpu.async_copy(h_hbm.at[sv[p].at[0, pl.ds(HCH, HCH)]], rb, gsb)
            accum_half(ra, 0)
            pltpu.make_async_copy(h_hbm.at[sv[p].at[0, pl.ds(HCH, HCH)]],
                                  rb, gsb).wait()

            @pl.when(k + 2 < nch)
            def _():
                fire_idx(k + 2, p)

            @pl.when(k + 1 < nch)
            def _():
                wait_idx_s(1 - p)
                fire_gather_a(1 - p)

            accum_half(rb, HCH // 16)

        def outer(kk, carry):
            k = kk * 2

            @pl.when(k < nch)
            def _():
                body(k, 0)

            @pl.when(k + 1 < nch)
            def _():
                body(k + 1, 1)

            return carry

        lax.fori_loop(0, (nch + 1) // 2, outer, 0)

        @pl.when(w < NTILES - 1)
        def _():
            pltpu.sync_copy(agg.at[pl.ds(0, BROWS)],
                            out_hbm.at[pl.ds(w * BROWS, BROWS)])

        @pl.when(w == NTILES - 1)
        def _():
            pltpu.sync_copy(agg.at[pl.ds(0, N - (NTILES - 1) * BROWS)],
                            out_hbm.at[pl.ds((NTILES - 1) * BROWS,
                                             N - (NTILES - 1) * BROWS)])

    return acc(h, ls, ld, zeros)


def _dense_body(h_ref, agg_ref, ws_ref, wn_ref, b_ref, out_ref):
    x = jnp.dot(h_ref[...], ws_ref[...], preferred_element_type=jnp.float32)
    x = x + jnp.dot(agg_ref[...], wn_ref[...], preferred_element_type=jnp.float32)
    out_ref[...] = jnp.maximum(x + b_ref[...], 0.0)


def _dense_update(h, agg, ws, wn, b2):
    br = 1000
    return pl.pallas_call(
        _dense_body,
        grid=(N // br,),
        in_specs=[
            pl.BlockSpec((br, D), lambda i: (i, 0)),
            pl.BlockSpec((br, D), lambda i: (i, 0)),
            pl.BlockSpec((D, D), lambda i: (0, 0)),
            pl.BlockSpec((D, D), lambda i: (0, 0)),
            pl.BlockSpec((1, D), lambda i: (0, 0)),
        ],
        out_specs=pl.BlockSpec((br, D), lambda i: (i, 0)),
        out_shape=jax.ShapeDtypeStruct((N, D), jnp.float32),
    )(h, agg, ws, wn, b2)


_PERM = [32 * j + 2 * i + t
         for j in range(D // 32)
         for t in range(2)
         for i in range(16)]


def kernel(h, edge_index, W_self, W_nei, b):
    src = edge_index[0].astype(jnp.int32)
    dst = edge_index[1].astype(jnp.int32)
    ls, ld = _partition(src, dst)
    zeros = jnp.zeros((AGG_ROWS, D), jnp.float32)
    W_nei_p = W_nei[:, jnp.asarray(_PERM, dtype=jnp.int32), :]
    for i in range(ITERS):
        hb = h.astype(jnp.bfloat16).reshape(N, D // 2, 2)
        h32 = lax.bitcast_convert_type(hb, jnp.int32)
        agg = _seg_sum(h32, ls, ld, zeros)
        h = _dense_update(h, agg, W_self[i], W_nei_p[i], b[i].reshape(1, D))
    return h


# speedup vs baseline: 1.2691x; 1.2691x over previous
import functools

import jax
import jax.numpy as jnp
from jax import lax
from jax.experimental import pallas as pl
from jax.experimental.pallas import tpu as pltpu
from jax.experimental.pallas import tpu_sc as plsc

N = 10000
D = 256
E = 160000
ITERS = 3

NTILES = 32
BROWS = 320
TRASH = 320
AGG_ROWS = 321

CHUNK = 128
HCH = CHUNK // 2
PC = 1280
NPC = E // PC
CCAP = CHUNK + 32
DATA0 = 128
LIST_LEN = DATA0 + E + CHUNK


def _partition(src, dst):
    mesh = plsc.VectorSubcoreMesh(core_axis_name="c", subcore_axis_name="s")

    @functools.partial(
        pl.kernel,
        out_type=[
            jax.ShapeDtypeStruct((NTILES, 1, LIST_LEN), jnp.int32),
            jax.ShapeDtypeStruct((NTILES, 1, LIST_LEN), jnp.int32),
        ],
        mesh=mesh,
        compiler_params=pltpu.CompilerParams(needs_layout_passes=False),
        scratch_types=[
            pltpu.VMEM((PC,), jnp.int32),
            pltpu.VMEM((PC,), jnp.int32),
            pltpu.VMEM((PC,), jnp.int32),
            pltpu.VMEM((PC,), jnp.int32),
            pltpu.VMEM((CCAP,), jnp.int32),
            pltpu.VMEM((CCAP,), jnp.int32),
            pltpu.VMEM((16,), jnp.int32),
            pltpu.SMEM((4,), jnp.int32),
            pltpu.SemaphoreType.DMA,
            pltpu.SemaphoreType.DMA,
            pltpu.SemaphoreType.DMA,
            pltpu.SemaphoreType.DMA,
        ],
    )
    def part(src_hbm, dst_hbm, ls_hbm, ld_hbm,
             sv0, sv1, dv0, dv1, cs, cd, hv, st,
             ss0, ss1, ds0, ds1):
        c = lax.axis_index("c")
        s = lax.axis_index("s")
        w = c * 16 + s
        lo = w * BROWS
        svr = (sv0, sv1)
        dvr = (dv0, dv1)
        ssem = (ss0, ss1)
        dsem = (ds0, ds1)
        st[0] = 0
        st[1] = DATA0
        st[2] = 0

        def fire_scan(k, p):
            pltpu.async_copy(src_hbm.at[pl.ds(k * PC, PC)], svr[p], ssem[p])
            pltpu.async_copy(dst_hbm.at[pl.ds(k * PC, PC)], dvr[p], dsem[p])

        fire_scan(0, 0)

        def scan_chunk(k, p):
            sv = svr[p]
            dv = dvr[p]
            pltpu.make_async_copy(src_hbm.at[pl.ds(0, PC)], sv, ssem[p]).wait()
            pltpu.make_async_copy(dst_hbm.at[pl.ds(0, PC)], dv, dsem[p]).wait()

            @pl.when(k + 1 < NPC)
            def _():
                fire_scan(k + 1, 1 - p)

            for g in range(PC // 32):
                d16a = dv[pl.ds(g * 32, 16)]
                d16b = dv[pl.ds(g * 32 + 16, 16)]
                ma = lax.shift_right_logical(d16a * 6554, 21) == w
                mb = lax.shift_right_logical(d16b * 6554, 21) == w
                anyc = plsc.all_reduce_population_count(ma | mb)[0]

                @pl.when(anyc > 0)
                def _():
                    off = st[0]
                    npa = plsc.all_reduce_population_count(ma)[0]
                    s16a = sv[pl.ds(g * 32, 16)]
                    s16b = sv[pl.ds(g * 32 + 16, 16)]
                    plsc.store_compressed(cs.at[pl.ds(off, 16)], s16a, mask=ma)
                    plsc.store_compressed(cd.at[pl.ds(off, 16)], d16a - lo,
                                          mask=ma)
                    npb = plsc.all_reduce_population_count(mb)[0]
                    off2 = off + npa
                    plsc.store_compressed(cs.at[pl.ds(off2, 16)], s16b, mask=mb)
                    plsc.store_compressed(cd.at[pl.ds(off2, 16)], d16b - lo,
                                          mask=mb)
                    st[0] = off2 + npb

                @pl.when(st[0] >= CHUNK)
                def _():
                    pos = pl.multiple_of(st[1], CHUNK)
                    pltpu.sync_copy(cs.at[pl.ds(0, CHUNK)],
                                    ls_hbm.at[w, 0, pl.ds(pos, CHUNK)])
                    pltpu.sync_copy(cd.at[pl.ds(0, CHUNK)],
                                    ld_hbm.at[w, 0, pl.ds(pos, CHUNK)])
                    for t in range(2):
                        ts = cs[pl.ds(CHUNK + t * 16, 16)]
                        cs[pl.ds(t * 16, 16)] = ts
                        td = cd[pl.ds(CHUNK + t * 16, 16)]
                        cd[pl.ds(t * 16, 16)] = td
                    st[0] = st[0] - CHUNK
                    st[1] = pos + CHUNK
                    st[2] = st[2] + 1

        def outer(kk, carry):
            scan_chunk(kk * 2, 0)

            @pl.when(kk * 2 + 1 < NPC)
            def _():
                scan_chunk(kk * 2 + 1, 1)

            return carry

        lax.fori_loop(0, (NPC + 1) // 2, outer, 0)

        off = st[0]
        iota = lax.iota(jnp.int32, 16)
        for g in range(CCAP // 16):
            keep = (iota + g * 16) < off
            vs = cs[pl.ds(g * 16, 16)]
            cs[pl.ds(g * 16, 16)] = jnp.where(keep, vs, 0)
            vd = cd[pl.ds(g * 16, 16)]
            cd[pl.ds(g * 16, 16)] = jnp.where(keep, vd, TRASH)
        pos = pl.multiple_of(st[1], CHUNK)
        pltpu.sync_copy(cs.at[pl.ds(0, CHUNK)],
                        ls_hbm.at[w, 0, pl.ds(pos, CHUNK)])
        pltpu.sync_copy(cd.at[pl.ds(0, CHUNK)],
                        ld_hbm.at[w, 0, pl.ds(pos, CHUNK)])
        nch = st[2] + 1
        hv[pl.ds(0, 16)] = jnp.full((16,), nch, jnp.int32)
        pltpu.sync_copy(hv.at[pl.ds(0, 8)], ls_hbm.at[w, 0, pl.ds(0, 8)])

    return part(src, dst)


def _seg_sum(h, ls, ld, zeros):
    mesh = plsc.VectorSubcoreMesh(core_axis_name="c", subcore_axis_name="s")

    @functools.partial(
        pl.kernel,
        out_type=jax.ShapeDtypeStruct((N, D), jnp.float32),
        mesh=mesh,
        compiler_params=pltpu.CompilerParams(needs_layout_passes=False),
        scratch_types=[
            pltpu.VMEM((16,), jnp.int32),
            pltpu.VMEM((1, CHUNK), jnp.int32),
            pltpu.VMEM((1, CHUNK), jnp.int32),
            pltpu.VMEM((CHUNK,), jnp.int32),
            pltpu.VMEM((CHUNK,), jnp.int32),
            pltpu.VMEM((CHUNK,), jnp.int32),
            pltpu.VMEM((HCH, D), jnp.float32),
            pltpu.VMEM((HCH, D), jnp.float32),
            pltpu.VMEM((AGG_ROWS, D), jnp.float32),
            pltpu.SemaphoreType.DMA,
            pltpu.SemaphoreType.DMA,
            pltpu.SemaphoreType.DMA,
            pltpu.SemaphoreType.DMA,
            pltpu.SemaphoreType.DMA,
            pltpu.SemaphoreType.DMA,
        ],
    )
    def acc(h_hbm, ls_hbm, ld_hbm, z_hbm, out_hbm,
            hv, sv0, sv1, dlv0, dlv1, dlw, ra, rb, agg,
            is0, is1, id0, id1, gsa, gsb):
        c = lax.axis_index("c")
        s = lax.axis_index("s")
        w = c * 16 + s
        sv = (sv0, sv1)
        dlv = (dlv0, dlv1)
        isem_s = (is0, is1)
        isem_d = (id0, id1)
        pltpu.sync_copy(z_hbm, agg)
        pltpu.sync_copy(ls_hbm.at[w, 0, pl.ds(0, 8)], hv.at[pl.ds(0, 8)])
        nch = hv[pl.ds(0, 16)][0]

        def fire_idx(k, p):
            base = pl.multiple_of(DATA0 + k * CHUNK, CHUNK)
            pltpu.async_copy(ls_hbm.at[w, 0, pl.ds(base, CHUNK)],
                             sv[p].at[0], isem_s[p])
            pltpu.async_copy(ld_hbm.at[w, 0, pl.ds(base, CHUNK)],
                             dlv[p], isem_d[p])

        def wait_idx_s(p):
            pltpu.make_async_copy(ls_hbm.at[w, 0, pl.ds(0, CHUNK)],
                                  sv[p].at[0], isem_s[p]).wait()

        def fire_gather_a(p):
            pltpu.async_copy(h_hbm.at[sv[p].at[0, pl.ds(0, HCH)]], ra, gsa)

        fire_idx(0, 0)

        @pl.when(nch > 1)
        def _():
            fire_idx(1, 1)

        wait_idx_s(0)
        fire_gather_a(0)

        def accum_half(rbuf, mbase):
            def edge_group(m, carry2):
                dl16 = dlw[pl.ds(mbase * 16 + m * 16, 16)]
                for i in range(0, 16, 2):
                    dla = dl16[i]
                    dlb = dl16[i + 1]
                    va = [rbuf[m * 16 + i, pl.ds(j * 16, 16)]
                          for j in range(D // 16)]
                    vb = [rbuf[m * 16 + i + 1, pl.ds(j * 16, 16)]
                          for j in range(D // 16)]
                    for j in range(D // 16):
                        plsc.addupdate(agg.at[dla, pl.ds(j * 16, 16)], va[j])
                    for j in range(D // 16):
                        plsc.addupdate(agg.at[dlb, pl.ds(j * 16, 16)], vb[j])
                return carry2

            lax.fori_loop(0, HCH // 16, edge_group, 0)

        def body(k, p):
            pltpu.make_async_copy(h_hbm.at[sv[p].at[0, pl.ds(0, HCH)]],
                                  ra, gsa).wait()
            pltpu.make_async_copy(ld_hbm.at[w, 0, pl.ds(0, CHUNK)],
                                  dlv[p], isem_d[p]).wait()
            for m in range(CHUNK // 16):
                dlw[pl.ds(m * 16, 16)] = dlv[p][pl.ds(m * 16, 16)]
            pltpu.async_copy(h_hbm.at[sv[p].at[0, pl.ds(HCH, HCH)]], rb, gsb)
            accum_half(ra, 0)
            pltpu.make_async_copy(h_hbm.at[sv[p].at[0, pl.ds(HCH, HCH)]],
                                  rb, gsb).wait()

            @pl.when(k + 2 < nch)
            def _():
                fire_idx(k + 2, p)

            @pl.when(k + 1 < nch)
            def _():
                wait_idx_s(1 - p)
                fire_gather_a(1 - p)

            accum_half(rb, HCH // 16)

        def outer(kk, carry):
            k = kk * 2

            @pl.when(k < nch)
            def _():
                body(k, 0)

            @pl.when(k + 1 < nch)
            def _():
                body(k + 1, 1)

            return carry

        lax.fori_loop(0, (nch + 1) // 2, outer, 0)

        @pl.when(w < NTILES - 1)
        def _():
            pltpu.sync_copy(agg.at[pl.ds(0, BROWS)],
                            out_hbm.at[pl.ds(w * BROWS, BROWS)])

        @pl.when(w == NTILES - 1)
        def _():
            pltpu.sync_copy(agg.at[pl.ds(0, N - (NTILES - 1) * BROWS)],
                            out_hbm.at[pl.ds((NTILES - 1) * BROWS,
                                             N - (NTILES - 1) * BROWS)])

    return acc(h, ls, ld, zeros)


def _dense_body(h_ref, agg_ref, ws_ref, wn_ref, b_ref, out_ref):
    x = jnp.dot(h_ref[...], ws_ref[...], preferred_element_type=jnp.float32)
    x = x + jnp.dot(agg_ref[...], wn_ref[...], preferred_element_type=jnp.float32)
    out_ref[...] = jnp.maximum(x + b_ref[...], 0.0)


def _dense_update(h, agg, ws, wn, b2):
    br = 1000
    return pl.pallas_call(
        _dense_body,
        grid=(N // br,),
        in_specs=[
            pl.BlockSpec((br, D), lambda i: (i, 0)),
            pl.BlockSpec((br, D), lambda i: (i, 0)),
            pl.BlockSpec((D, D), lambda i: (0, 0)),
            pl.BlockSpec((D, D), lambda i: (0, 0)),
            pl.BlockSpec((1, D), lambda i: (0, 0)),
        ],
        out_specs=pl.BlockSpec((br, D), lambda i: (i, 0)),
        out_shape=jax.ShapeDtypeStruct((N, D), jnp.float32),
    )(h, agg, ws, wn, b2)


def kernel(h, edge_index, W_self, W_nei, b):
    src = edge_index[0].astype(jnp.int32)
    dst = edge_index[1].astype(jnp.int32)
    ls, ld = _partition(src, dst)
    zeros = jnp.zeros((AGG_ROWS, D), jnp.float32)
    for i in range(ITERS):
        agg = _seg_sum(h, ls, ld, zeros)
        h = _dense_update(h, agg, W_self[i], W_nei[i], b[i].reshape(1, D))
    return h
